# baseline (speedup 1.0000x reference)
import functools

import jax
import jax.numpy as jnp
from jax.experimental import pallas as pl
from jax.experimental.pallas import tpu as pltpu

DIM = 4096
NUM_EXPERTS = 64
TOP_K = 8
TOKEN_BLOCK = 1024
KSPLIT = DIM // 2


def _gate_body(xa_ref, xb_ref, wta_ref, wtb_ref, w_out_ref, i_out_ref):
    dims = (((1,), (0,)), ((), ()))
    logits = jax.lax.dot_general(
        xa_ref[...], wta_ref[...], dimension_numbers=dims,
        preferred_element_type=jnp.float32,
    ) + jax.lax.dot_general(
        xb_ref[...], wtb_ref[...], dimension_numbers=dims,
        preferred_element_type=jnp.float32,
    )
    tb = logits.shape[0]
    lt = logits.T
    row = jax.lax.broadcasted_iota(jnp.int32, lt.shape, 0)
    row8 = jax.lax.broadcasted_iota(jnp.int32, (TOP_K, tb), 0)
    neg_inf = jnp.float32(float("-inf"))

    work = lt
    top_v = jnp.zeros((TOP_K, tb), jnp.float32)
    top_i = jnp.zeros((TOP_K, tb), jnp.int32)
    for k in range(TOP_K):
        m = jnp.max(work, axis=0, keepdims=True)
        idx = jnp.min(jnp.where(work == m, row, NUM_EXPERTS), axis=0,
                      keepdims=True)
        top_v = jnp.where(row8 == k, m, top_v)
        top_i = jnp.where(row8 == k, idx, top_i)
        work = jnp.where(row == idx, neg_inf, work)

    m0 = jnp.max(top_v, axis=0, keepdims=True)
    e = jnp.exp(top_v - m0)
    w_out_ref[...] = (e / jnp.sum(e, axis=0, keepdims=True)).T
    i_out_ref[...] = top_i.T


@functools.partial(jax.jit, static_argnames=("interpret",))
def kernel(x, W, interpret=False):
    b, n, d = x.shape
    tokens = b * n
    xt = x.reshape(tokens, d)
    wt = W.T
    grid = (tokens // TOKEN_BLOCK,)
    weights, indices = pl.pallas_call(
        _gate_body,
        grid=grid,
        in_specs=[
            pl.BlockSpec((TOKEN_BLOCK, KSPLIT), lambda i: (i, 0)),
            pl.BlockSpec((TOKEN_BLOCK, KSPLIT), lambda i: (i, 1)),
            pl.BlockSpec((KSPLIT, NUM_EXPERTS), lambda i: (0, 0)),
            pl.BlockSpec((KSPLIT, NUM_EXPERTS), lambda i: (1, 0)),
        ],
        out_specs=[
            pl.BlockSpec((TOKEN_BLOCK, TOP_K), lambda i: (i, 0)),
            pl.BlockSpec((TOKEN_BLOCK, TOP_K), lambda i: (i, 0)),
        ],
        out_shape=[
            jax.ShapeDtypeStruct((tokens, TOP_K), jnp.float32),
            jax.ShapeDtypeStruct((tokens, TOP_K), jnp.int32),
        ],
        compiler_params=pltpu.CompilerParams(
            dimension_semantics=("arbitrary",),
        ),
        interpret=interpret,
    )(xt, xt, wt, wt)
    return weights.reshape(b, n, TOP_K), indices.reshape(b, n, TOP_K)


# speedup vs baseline: 1.0007x; 1.0007x over previous
import functools

import jax
import jax.numpy as jnp
from jax.experimental import pallas as pl
from jax.experimental.pallas import tpu as pltpu

DIM = 4096
NUM_EXPERTS = 64
TOP_K = 8
TOKEN_BLOCK = 1024


def _gate_body(x_ref, wt_ref, w_out_ref, i_out_ref):
    logits = jax.lax.dot_general(
        x_ref[...], wt_ref[...],
        dimension_numbers=(((1,), (0,)), ((), ())),
        preferred_element_type=jnp.float32,
    )
    tb = logits.shape[0]
    lt = logits.T
    row = jax.lax.broadcasted_iota(jnp.int32, lt.shape, 0)
    row8 = jax.lax.broadcasted_iota(jnp.int32, (TOP_K, tb), 0)
    neg_inf = jnp.float32(float("-inf"))

    work = lt
    top_v = jnp.zeros((TOP_K, tb), jnp.float32)
    top_i = jnp.zeros((TOP_K, tb), jnp.int32)
    for k in range(TOP_K):
        m = jnp.max(work, axis=0, keepdims=True)
        idx = jnp.min(jnp.where(work == m, row, NUM_EXPERTS), axis=0,
                      keepdims=True)
        top_v = jnp.where(row8 == k, m, top_v)
        top_i = jnp.where(row8 == k, idx, top_i)
        work = jnp.where(row == idx, neg_inf, work)

    m0 = jnp.max(top_v, axis=0, keepdims=True)
    e = jnp.exp(top_v - m0)
    w_out_ref[...] = (e / jnp.sum(e, axis=0, keepdims=True)).T
    i_out_ref[...] = top_i.T


@functools.partial(jax.jit, static_argnames=("interpret",))
def kernel(x, W, interpret=False):
    b, n, d = x.shape
    tokens = b * n
    xt = x.reshape(tokens, d)
    wt = W.T
    grid = (tokens // TOKEN_BLOCK,)
    weights, indices = pl.pallas_call(
        _gate_body,
        grid=grid,
        in_specs=[
            pl.BlockSpec((TOKEN_BLOCK, d), lambda i: (i, 0)),
            pl.BlockSpec((d, NUM_EXPERTS), lambda i: (0, 0)),
        ],
        out_specs=[
            pl.BlockSpec((TOKEN_BLOCK, TOP_K), lambda i: (i, 0)),
            pl.BlockSpec((TOKEN_BLOCK, TOP_K), lambda i: (i, 0)),
        ],
        out_shape=[
            jax.ShapeDtypeStruct((tokens, TOP_K), jnp.float32),
            jax.ShapeDtypeStruct((tokens, TOP_K), jnp.int32),
        ],
        compiler_params=pltpu.CompilerParams(
            dimension_semantics=("arbitrary",),
        ),
        interpret=interpret,
    )(xt, wt)
    return weights.reshape(b, n, TOP_K), indices.reshape(b, n, TOP_K)


# speedup vs baseline: 1.0768x; 1.0761x over previous
import functools

import jax
import jax.numpy as jnp
from jax.experimental import pallas as pl
from jax.experimental.pallas import tpu as pltpu

DIM = 4096
NUM_EXPERTS = 64
TOP_K = 8
TOKEN_BLOCK = 512


def _topk_softmax(logits):
    tb = logits.shape[0]
    lt = logits.T
    row = jax.lax.broadcasted_iota(jnp.int32, lt.shape, 0)
    row8 = jax.lax.broadcasted_iota(jnp.int32, (TOP_K, tb), 0)
    neg_inf = jnp.float32(float("-inf"))

    work = lt
    top_v = jnp.zeros((TOP_K, tb), jnp.float32)
    top_i = jnp.zeros((TOP_K, tb), jnp.int32)
    for k in range(TOP_K):
        m = jnp.max(work, axis=0, keepdims=True)
        idx = jnp.min(jnp.where(work == m, row, NUM_EXPERTS), axis=0,
                      keepdims=True)
        top_v = jnp.where(row8 == k, m, top_v)
        top_i = jnp.where(row8 == k, idx, top_i)
        work = jnp.where(row == idx, neg_inf, work)

    m0 = jnp.max(top_v, axis=0, keepdims=True)
    e = jnp.exp(top_v - m0)
    return e / jnp.sum(e, axis=0, keepdims=True), top_i


def _gate_body(xa_ref, xb_ref, wt_ref, wa_ref, ia_ref, wb_ref, ib_ref):
    dims = (((1,), (0,)), ((), ()))
    la = jax.lax.dot_general(xa_ref[...], wt_ref[...], dimension_numbers=dims,
                             preferred_element_type=jnp.float32)
    wa_ref[...], ia_ref[...] = _topk_softmax(la)
    lb = jax.lax.dot_general(xb_ref[...], wt_ref[...], dimension_numbers=dims,
                             preferred_element_type=jnp.float32)
    wb_ref[...], ib_ref[...] = _topk_softmax(lb)


@functools.partial(jax.jit, static_argnames=("interpret",))
def kernel(x, W, interpret=False):
    b, n, d = x.shape
    tokens = b * n
    half = tokens // 2
    xt = x.reshape(tokens, d)
    wt = W.T
    grid = (tokens // (2 * TOKEN_BLOCK),)
    w_e, i_e, w_o, i_o = pl.pallas_call(
        _gate_body,
        grid=grid,
        in_specs=[
            pl.BlockSpec((TOKEN_BLOCK, d), lambda i: (2 * i, 0)),
            pl.BlockSpec((TOKEN_BLOCK, d), lambda i: (2 * i + 1, 0)),
            pl.BlockSpec((d, NUM_EXPERTS), lambda i: (0, 0)),
        ],
        out_specs=[
            pl.BlockSpec((TOP_K, TOKEN_BLOCK), lambda i: (0, i)),
            pl.BlockSpec((TOP_K, TOKEN_BLOCK), lambda i: (0, i)),
            pl.BlockSpec((TOP_K, TOKEN_BLOCK), lambda i: (0, i)),
            pl.BlockSpec((TOP_K, TOKEN_BLOCK), lambda i: (0, i)),
        ],
        out_shape=[
            jax.ShapeDtypeStruct((TOP_K, half), jnp.float32),
            jax.ShapeDtypeStruct((TOP_K, half), jnp.int32),
            jax.ShapeDtypeStruct((TOP_K, half), jnp.float32),
            jax.ShapeDtypeStruct((TOP_K, half), jnp.int32),
        ],
        compiler_params=pltpu.CompilerParams(
            dimension_semantics=("arbitrary",),
        ),
        interpret=interpret,
    )(xt, xt, wt)

    def _merge(even, odd, dtype):
        nb = half // TOKEN_BLOCK
        e_r = even.reshape(TOP_K, nb, TOKEN_BLOCK)
        o_r = odd.reshape(TOP_K, nb, TOKEN_BLOCK)
        full = jnp.stack([e_r, o_r], axis=2).reshape(TOP_K, tokens)
        return full.T.reshape(b, n, TOP_K).astype(dtype)

    return _merge(w_e, w_o, jnp.float32), _merge(i_e, i_o, jnp.int32)
